# baseline (speedup 1.0000x reference)
import functools

import jax
import jax.numpy as jnp
from jax import lax
from jax.experimental import pallas as pl
from jax.experimental.pallas import tpu as pltpu
from jax.experimental.pallas import tpu_sc as plsc


def _make_sc_mlp_gather(B, D, NC, NS, off):
    NW = NC * NS
    per_w = B // NW
    CH = 64
    n_ch = per_w // CH

    mesh = plsc.VectorSubcoreMesh(core_axis_name="c", subcore_axis_name="s")

    def buf_set():
        return (
            pltpu.VMEM((CH,), jnp.int32),
            pltpu.VMEM((CH,), jnp.int32),
            pltpu.VMEM((CH, D), jnp.float32),
            pltpu.VMEM((CH, D), jnp.float32),
            pltpu.SemaphoreType.DMA,
            pltpu.SemaphoreType.DMA,
        )

    @functools.partial(
        pl.kernel,
        out_type=(
            jax.ShapeDtypeStruct((B, D), jnp.float32),
            jax.ShapeDtypeStruct((B, D), jnp.float32),
        ),
        mesh=mesh,
        scratch_types=buf_set() + buf_set(),
    )
    def sc_mlp(users_hbm, items_hbm, mu_t, mi_t,
               mlp_u_out, mlp_i_out, *bufs):
        wid = lax.axis_index("s") * NC + lax.axis_index("c")
        sets = (bufs[:6], bufs[6:])

        def issue(c):
            uidx, iidx, mu, mi, sem_g, _ = sets[c % 2]
            base = wid * per_w + c * CH
            pltpu.sync_copy(users_hbm.at[pl.ds(off + base, CH)], uidx)
            pltpu.sync_copy(items_hbm.at[pl.ds(off + base, CH)], iidx)
            return (pltpu.async_copy(mu_t.at[uidx], mu, sem_g),
                    pltpu.async_copy(mi_t.at[iidx], mi, sem_g))

        pending = {0: issue(0)}
        if n_ch > 1:
            pending[1] = issue(1)
        outcps = {}

        for c in range(n_ch):
            _, _, mu, mi, _, sem_o = sets[c % 2]
            base = wid * per_w + c * CH
            for cp in pending.pop(c):
                cp.wait()
            outcps[c] = (
                pltpu.async_copy(mu, mlp_u_out.at[pl.ds(base, CH)], sem_o),
                pltpu.async_copy(mi, mlp_i_out.at[pl.ds(base, CH)], sem_o),
            )
            nxt = c + 2
            if nxt < n_ch:
                for cp in outcps.pop(c):
                    cp.wait()
                pending[nxt] = issue(nxt)

        for c in sorted(outcps):
            for cp in outcps[c]:
                cp.wait()

    return sc_mlp


def _tc_mlp_body(mu_ref, mi_ref, w1u_ref, w1i_ref, b1_ref,
                 w2_ref, b2_ref, w3_ref, b3_ref, wnm_ref, bn_ref, out_ref):
    h = jnp.dot(mu_ref[...], w1u_ref[...], preferred_element_type=jnp.float32)
    h = h + jnp.dot(mi_ref[...], w1i_ref[...], preferred_element_type=jnp.float32)
    h = jax.nn.relu(h + b1_ref[...])
    h = jax.nn.relu(jnp.dot(h, w2_ref[...], preferred_element_type=jnp.float32)
                    + b2_ref[...])
    h = jax.nn.relu(jnp.dot(h, w3_ref[...], preferred_element_type=jnp.float32)
                    + b3_ref[...])
    out_ref[...] = h * wnm_ref[...] + bn_ref[...]


def _make_tc_mlp(B, D, H1, H2, H3):
    BLK = 2048
    grid = (B // BLK,)
    full = lambda shape: pl.BlockSpec(shape, lambda i: (0, 0))
    return pl.pallas_call(
        _tc_mlp_body,
        grid=grid,
        in_specs=[
            pl.BlockSpec((BLK, D), lambda i: (i, 0)),
            pl.BlockSpec((BLK, D), lambda i: (i, 0)),
            full((D, H1)),
            full((D, H1)),
            full((1, H1)),
            full((H1, H2)),
            full((1, H2)),
            full((H2, H3)),
            full((1, H3)),
            full((1, H3)),
            full((1, H3)),
        ],
        out_specs=pl.BlockSpec((BLK, H3), lambda i: (i, 0)),
        out_shape=jax.ShapeDtypeStruct((B, H3), jnp.float32),
    )


def _make_sc_final(B, D, NC, NS, WN, off):
    NW = NC * NS
    per_w = B // NW
    CH = 64
    n_ch = per_w // CH
    K8 = D // 16

    mesh = plsc.VectorSubcoreMesh(core_axis_name="c", subcore_axis_name="s")

    def buf_set():
        return (
            pltpu.VMEM((CH,), jnp.int32),
            pltpu.VMEM((CH,), jnp.int32),
            pltpu.VMEM((CH, D), jnp.float32),
            pltpu.VMEM((CH, D), jnp.float32),
            pltpu.VMEM((CH, 16), jnp.float32),
            pltpu.SemaphoreType.DMA,
        )

    @functools.partial(
        pl.kernel,
        out_type=jax.ShapeDtypeStruct((B,), jnp.float32),
        mesh=mesh,
        scratch_types=(
            pltpu.VMEM((WN,), jnp.float32),
            pltpu.VMEM((per_w,), jnp.float32),
        ) + buf_set() + buf_set(),
    )
    def sc_final(users_hbm, items_hbm, gu_t, gi_t, wn_hbm, fin_hbm,
                 out_hbm, wnv, outv, *bufs):
        wid = lax.axis_index("s") * NC + lax.axis_index("c")
        pltpu.sync_copy(wn_hbm, wnv)
        sets = (bufs[:6], bufs[6:])
        lane = jax.lax.iota(jnp.int32, 16)
        perms = [lane ^ k for k in (8, 4, 2, 1)]

        def issue(c):
            uidx, iidx, gu, gi, finv, sem_g = sets[c % 2]
            base = wid * per_w + c * CH
            pltpu.sync_copy(users_hbm.at[pl.ds(off + base, CH)], uidx)
            pltpu.sync_copy(items_hbm.at[pl.ds(off + base, CH)], iidx)
            return (pltpu.async_copy(gu_t.at[uidx], gu, sem_g),
                    pltpu.async_copy(gi_t.at[iidx], gi, sem_g),
                    pltpu.async_copy(fin_hbm.at[pl.ds(base, CH)], finv, sem_g))

        pending = {0: issue(0)}
        if n_ch > 1:
            pending[1] = issue(1)

        for c in range(n_ch):
            _, _, gu, gi, finv, _ = sets[c % 2]
            for cp in pending.pop(c):
                cp.wait()

            def grp(q, carry):
                def row(j, acc):
                    i = q * 16 + j
                    v = finv[i, :] + (gu[i, pl.ds(0, 16)]
                                      * gi[i, pl.ds(0, 16)]
                                      * wnv[pl.ds(0, 16)])
                    for k in range(1, K8):
                        v = v + (gu[i, pl.ds(k * 16, 16)]
                                 * gi[i, pl.ds(k * 16, 16)]
                                 * wnv[pl.ds(k * 16, 16)])
                    for p in perms:
                        v = v + jnp.take(v, p)
                    return jnp.where(lane == j, v, acc)

                acc = lax.fori_loop(0, 16, row, jnp.zeros(16, jnp.float32))
                outv[pl.ds(c * CH + q * 16, 16)] = 1.0 / (1.0 + jnp.exp(-acc))
                return carry

            lax.fori_loop(0, CH // 16, grp, 0)
            nxt = c + 2
            if nxt < n_ch:
                pending[nxt] = issue(nxt)

        pltpu.sync_copy(outv, out_hbm.at[pl.ds(wid * per_w, per_w)])

    return sc_final


def kernel(gmf_user_table, gmf_item_table, mlp_user_table, mlp_item_table,
           W1, b1, W2, b2, W3, b3, Wn, bn, users, items):
    B = users.shape[0]
    D = gmf_user_table.shape[1]
    H1, H2, H3 = W1.shape[1], W2.shape[1], W3.shape[1]

    info = plsc.get_sparse_core_info()
    NC, NS = info.num_cores, info.num_subcores

    users = users.astype(jnp.int32)
    items = items.astype(jnp.int32)
    wn_flat = Wn.reshape(-1)
    wn_m = Wn[D:, :].reshape(1, H3)
    bn16 = jnp.broadcast_to(bn.reshape(1, 1) / H3, (1, H3))

    n_split = 2
    H = B // n_split
    tc = _make_tc_mlp(H, D, H1, H2, H3)

    mlp_rows = [_make_sc_mlp_gather(H, D, NC, NS, i * H)(
                    users, items, mlp_user_table, mlp_item_table)
                for i in range(n_split)]
    fins = [tc(mu_g, mi_g,
               W1[:D], W1[D:], b1.reshape(1, H1),
               W2, b2.reshape(1, H2),
               W3, b3.reshape(1, H3),
               wn_m, bn16)
            for (mu_g, mi_g) in mlp_rows]
    preds = [_make_sc_final(H, D, NC, NS, wn_flat.shape[0], i * H)(
                 users, items, gmf_user_table, gmf_item_table,
                 wn_flat, fins[i])
             for i in range(n_split)]
    return jnp.concatenate(preds, axis=0)


# speedup vs baseline: 1.0128x; 1.0128x over previous
import functools

import jax
import jax.numpy as jnp
from jax import lax
from jax.experimental import pallas as pl
from jax.experimental.pallas import tpu as pltpu
from jax.experimental.pallas import tpu_sc as plsc


def _make_sc_gather(B, D, NC, NS, WN, off):
    NW = NC * NS
    per_w = B // NW
    CH = 64
    n_ch = per_w // CH
    K8 = D // 16

    mesh = plsc.VectorSubcoreMesh(core_axis_name="c", subcore_axis_name="s")

    def buf_set():
        return (
            pltpu.VMEM((CH,), jnp.int32),
            pltpu.VMEM((CH,), jnp.int32),
            pltpu.VMEM((CH, D), jnp.float32),
            pltpu.VMEM((CH, D), jnp.float32),
            pltpu.VMEM((CH, D), jnp.float32),
            pltpu.VMEM((CH, D), jnp.float32),
            pltpu.VMEM((CH * 16,), jnp.float32),
            pltpu.SemaphoreType.DMA,
            pltpu.SemaphoreType.DMA,
        )

    @functools.partial(
        pl.kernel,
        out_type=(
            jax.ShapeDtypeStruct((B, D), jnp.float32),
            jax.ShapeDtypeStruct((B, D), jnp.float32),
            jax.ShapeDtypeStruct((B * 16,), jnp.float32),
        ),
        mesh=mesh,
        scratch_types=(
            pltpu.VMEM((WN,), jnp.float32),
        ) + buf_set() + buf_set(),
    )
    def sc_gather(users_hbm, items_hbm, gu_t, gi_t, mu_t, mi_t, wn_hbm,
                  mlp_u_out, mlp_i_out, gmfp_out, wnv, *bufs):
        wid = lax.axis_index("s") * NC + lax.axis_index("c")
        pltpu.sync_copy(wn_hbm, wnv)
        sets = (bufs[:9], bufs[9:])

        def issue(c):
            uidx, iidx, gu, gi, mu, mi, _, sem_g, _ = sets[c % 2]
            base = wid * per_w + c * CH
            pltpu.sync_copy(users_hbm.at[pl.ds(off + base, CH)], uidx)
            pltpu.sync_copy(items_hbm.at[pl.ds(off + base, CH)], iidx)
            return (pltpu.async_copy(gu_t.at[uidx], gu, sem_g),
                    pltpu.async_copy(gi_t.at[iidx], gi, sem_g),
                    pltpu.async_copy(mu_t.at[uidx], mu, sem_g),
                    pltpu.async_copy(mi_t.at[iidx], mi, sem_g))

        pending = {0: issue(0)}
        if n_ch > 1:
            pending[1] = issue(1)
        outcps = {}

        for c in range(n_ch):
            _, _, gu, gi, mu, mi, gmfp, _, sem_o = sets[c % 2]
            base = wid * per_w + c * CH
            for cp in pending.pop(c):
                cp.wait()
            o1 = pltpu.async_copy(mu, mlp_u_out.at[pl.ds(base, CH)], sem_o)
            o2 = pltpu.async_copy(mi, mlp_i_out.at[pl.ds(base, CH)], sem_o)

            def row_body(i, carry):
                acc = gu[i, pl.ds(0, 16)] * gi[i, pl.ds(0, 16)] * wnv[pl.ds(0, 16)]
                for k in range(1, K8):
                    acc = acc + (gu[i, pl.ds(k * 16, 16)]
                                 * gi[i, pl.ds(k * 16, 16)]
                                 * wnv[pl.ds(k * 16, 16)])
                gmfp[pl.ds(i * 16, 16)] = acc
                return carry

            lax.fori_loop(0, CH, row_body, 0)
            o3 = pltpu.async_copy(gmfp, gmfp_out.at[pl.ds(base * 16, CH * 16)],
                                  sem_o)
            outcps[c] = (o1, o2, o3)

            nxt = c + 2
            if nxt < n_ch:
                for cp in outcps.pop(c):
                    cp.wait()
                pending[nxt] = issue(nxt)

        for c in sorted(outcps):
            for cp in outcps[c]:
                cp.wait()

    return sc_gather


def _tc_mlp_body(mu_ref, mi_ref, w1u_ref, w1i_ref, b1_ref,
                 w2_ref, b2_ref, w3_ref, b3_ref, wnm_ref, bn_ref, out_ref):
    h = jnp.dot(mu_ref[...], w1u_ref[...], preferred_element_type=jnp.float32)
    h = h + jnp.dot(mi_ref[...], w1i_ref[...], preferred_element_type=jnp.float32)
    h = jax.nn.relu(h + b1_ref[...])
    h = jax.nn.relu(jnp.dot(h, w2_ref[...], preferred_element_type=jnp.float32)
                    + b2_ref[...])
    h = jax.nn.relu(jnp.dot(h, w3_ref[...], preferred_element_type=jnp.float32)
                    + b3_ref[...])
    out_ref[...] = h * wnm_ref[...] + bn_ref[...]


def _make_tc_mlp(B, D, H1, H2, H3):
    BLK = 1024
    grid = (B // BLK,)
    full = lambda shape: pl.BlockSpec(shape, lambda i: (0, 0))
    return pl.pallas_call(
        _tc_mlp_body,
        grid=grid,
        in_specs=[
            pl.BlockSpec((BLK, D), lambda i: (i, 0)),
            pl.BlockSpec((BLK, D), lambda i: (i, 0)),
            full((D, H1)),
            full((D, H1)),
            full((1, H1)),
            full((H1, H2)),
            full((1, H2)),
            full((H2, H3)),
            full((1, H3)),
            full((1, H3)),
            full((1, H3)),
        ],
        out_specs=pl.BlockSpec((BLK, H3), lambda i: (i, 0)),
        out_shape=jax.ShapeDtypeStruct((B, H3), jnp.float32),
    )


def _make_sc_final(B, NC, NS):
    NW = NC * NS
    half = B // 2
    per_t = half // (NW // 2)
    G = per_t // 16

    mesh = plsc.VectorSubcoreMesh(core_axis_name="c", subcore_axis_name="s")

    @functools.partial(
        pl.kernel,
        out_type=jax.ShapeDtypeStruct((B,), jnp.float32),
        mesh=mesh,
        scratch_types=(
            pltpu.VMEM((per_t, 16), jnp.float32),
            pltpu.VMEM((per_t * 16,), jnp.float32),
            pltpu.VMEM((per_t,), jnp.float32),
        ),
    )
    def sc_final(fin0, fin1, gmf0, gmf1, out_hbm, finv, gmfv, outv):
        wid = lax.axis_index("s") * NC + lax.axis_index("c")
        lane = jax.lax.iota(jnp.int32, 16)
        perms = [lane ^ k for k in (8, 4, 2, 1)]

        def work(fin_ref, gmf_ref, tid, out_base):
            row0 = tid * per_t
            pltpu.sync_copy(fin_ref.at[pl.ds(row0, per_t)], finv)
            pltpu.sync_copy(gmf_ref.at[pl.ds(row0 * 16, per_t * 16)], gmfv)

            def grp(q, carry):
                def row(j, acc):
                    i = q * 16 + j
                    v = finv[i, :] + gmfv[pl.ds(i * 16, 16)]
                    for p in perms:
                        v = v + jnp.take(v, p)
                    return jnp.where(lane == j, v, acc)

                acc = lax.fori_loop(0, 16, row, jnp.zeros(16, jnp.float32))
                outv[pl.ds(q * 16, 16)] = 1.0 / (1.0 + jnp.exp(-acc))
                return carry

            lax.fori_loop(0, G, grp, 0)
            pltpu.sync_copy(outv, out_hbm.at[pl.ds(out_base + row0, per_t)])

        @pl.when(wid < NW // 2)
        def _():
            work(fin0, gmf0, wid, 0)

        @pl.when(wid >= NW // 2)
        def _():
            work(fin1, gmf1, wid - NW // 2, half)

    return sc_final


def kernel(gmf_user_table, gmf_item_table, mlp_user_table, mlp_item_table,
           W1, b1, W2, b2, W3, b3, Wn, bn, users, items):
    B = users.shape[0]
    D = gmf_user_table.shape[1]
    H1, H2, H3 = W1.shape[1], W2.shape[1], W3.shape[1]

    info = plsc.get_sparse_core_info()
    NC, NS = info.num_cores, info.num_subcores

    users = users.astype(jnp.int32)
    items = items.astype(jnp.int32)
    wn_flat = Wn.reshape(-1)
    wn_m = Wn[D:, :].reshape(1, H3)
    bn16 = jnp.broadcast_to(bn.reshape(1, 1) / H3, (1, H3))

    n_split = 2
    H = B // n_split
    tc = _make_tc_mlp(H, D, H1, H2, H3)

    sc_outs = [_make_sc_gather(H, D, NC, NS, wn_flat.shape[0], i * H)(
                   users, items,
                   gmf_user_table, gmf_item_table,
                   mlp_user_table, mlp_item_table, wn_flat)
               for i in range(n_split)]
    fins = [tc(mu_g, mi_g,
               W1[:D], W1[D:], b1.reshape(1, H1),
               W2, b2.reshape(1, H2),
               W3, b3.reshape(1, H3),
               wn_m, bn16)
            for (mu_g, mi_g, _) in sc_outs]
    fin = _make_sc_final(B, NC, NS)
    return fin(fins[0], fins[1], sc_outs[0][2], sc_outs[1][2])


# speedup vs baseline: 1.0677x; 1.0543x over previous
import functools

import jax
import jax.numpy as jnp
from jax import lax
from jax.experimental import pallas as pl
from jax.experimental.pallas import tpu as pltpu
from jax.experimental.pallas import tpu_sc as plsc


def _make_sc_gather(B, D, NC, NS, WN, off):
    NW = NC * NS
    per_w = B // NW
    CH = 64
    n_ch = per_w // CH
    K8 = D // 16

    mesh = plsc.VectorSubcoreMesh(core_axis_name="c", subcore_axis_name="s")

    def buf_set():
        return (
            pltpu.VMEM((CH, D), jnp.float32),
            pltpu.VMEM((CH, D), jnp.float32),
            pltpu.VMEM((CH, D), jnp.float32),
            pltpu.VMEM((CH, D), jnp.float32),
            pltpu.VMEM((CH * 16,), jnp.float32),
            pltpu.SemaphoreType.DMA,
            pltpu.SemaphoreType.DMA,
        )

    @functools.partial(
        pl.kernel,
        out_type=(
            jax.ShapeDtypeStruct((B, D), jnp.float32),
            jax.ShapeDtypeStruct((B, D), jnp.float32),
            jax.ShapeDtypeStruct((B * 16,), jnp.float32),
        ),
        mesh=mesh,
        scratch_types=(
            pltpu.VMEM((WN,), jnp.float32),
            pltpu.VMEM((per_w,), jnp.int32),
            pltpu.VMEM((per_w,), jnp.int32),
        ) + buf_set() + buf_set(),
    )
    def sc_gather(users_hbm, items_hbm, gu_t, gi_t, mu_t, mi_t, wn_hbm,
                  mlp_u_out, mlp_i_out, gmfp_out, wnv, uidx_all, iidx_all,
                  *bufs):
        wid = lax.axis_index("s") * NC + lax.axis_index("c")
        pltpu.sync_copy(users_hbm.at[pl.ds(off + wid * per_w, per_w)],
                        uidx_all)
        pltpu.sync_copy(items_hbm.at[pl.ds(off + wid * per_w, per_w)],
                        iidx_all)
        pltpu.sync_copy(wn_hbm, wnv)
        sets = (bufs[:7], bufs[7:])

        def issue(c):
            gu, gi, mu, mi, _, sem_g, _ = sets[c % 2]
            uidx = uidx_all.at[pl.ds(c * CH, CH)]
            iidx = iidx_all.at[pl.ds(c * CH, CH)]
            return (pltpu.async_copy(gu_t.at[uidx], gu, sem_g),
                    pltpu.async_copy(gi_t.at[iidx], gi, sem_g),
                    pltpu.async_copy(mu_t.at[uidx], mu, sem_g),
                    pltpu.async_copy(mi_t.at[iidx], mi, sem_g))

        pending = {0: issue(0)}
        if n_ch > 1:
            pending[1] = issue(1)
        outcps = {}

        for c in range(n_ch):
            gu, gi, mu, mi, gmfp, _, sem_o = sets[c % 2]
            base = wid * per_w + c * CH
            for cp in pending.pop(c):
                cp.wait()
            o1 = pltpu.async_copy(mu, mlp_u_out.at[pl.ds(base, CH)], sem_o)
            o2 = pltpu.async_copy(mi, mlp_i_out.at[pl.ds(base, CH)], sem_o)

            def row_body(i, carry):
                acc = gu[i, pl.ds(0, 16)] * gi[i, pl.ds(0, 16)] * wnv[pl.ds(0, 16)]
                for k in range(1, K8):
                    acc = acc + (gu[i, pl.ds(k * 16, 16)]
                                 * gi[i, pl.ds(k * 16, 16)]
                                 * wnv[pl.ds(k * 16, 16)])
                gmfp[pl.ds(i * 16, 16)] = acc
                return carry

            lax.fori_loop(0, CH, row_body, 0)
            o3 = pltpu.async_copy(gmfp, gmfp_out.at[pl.ds(base * 16, CH * 16)],
                                  sem_o)
            outcps[c] = (o1, o2, o3)

            nxt = c + 2
            if nxt < n_ch:
                for cp in outcps.pop(c):
                    cp.wait()
                pending[nxt] = issue(nxt)

        for c in sorted(outcps):
            for cp in outcps[c]:
                cp.wait()

    return sc_gather


def _tc_mlp_body(mu_ref, mi_ref, w1u_ref, w1i_ref, b1_ref,
                 w2_ref, b2_ref, w3_ref, b3_ref, wnm_ref, bn_ref, out_ref):
    h = jnp.dot(mu_ref[...], w1u_ref[...], preferred_element_type=jnp.float32)
    h = h + jnp.dot(mi_ref[...], w1i_ref[...], preferred_element_type=jnp.float32)
    h = jax.nn.relu(h + b1_ref[...])
    h = jax.nn.relu(jnp.dot(h, w2_ref[...], preferred_element_type=jnp.float32)
                    + b2_ref[...])
    h = jax.nn.relu(jnp.dot(h, w3_ref[...], preferred_element_type=jnp.float32)
                    + b3_ref[...])
    out_ref[...] = h * wnm_ref[...] + bn_ref[...]


def _make_tc_mlp(B, D, H1, H2, H3):
    BLK = 2048
    grid = (B // BLK,)
    full = lambda shape: pl.BlockSpec(shape, lambda i: (0, 0))
    return pl.pallas_call(
        _tc_mlp_body,
        grid=grid,
        in_specs=[
            pl.BlockSpec((BLK, D), lambda i: (i, 0)),
            pl.BlockSpec((BLK, D), lambda i: (i, 0)),
            full((D, H1)),
            full((D, H1)),
            full((1, H1)),
            full((H1, H2)),
            full((1, H2)),
            full((H2, H3)),
            full((1, H3)),
            full((1, H3)),
            full((1, H3)),
        ],
        out_specs=pl.BlockSpec((BLK, H3), lambda i: (i, 0)),
        out_shape=jax.ShapeDtypeStruct((B, H3), jnp.float32),
    )


def _make_sc_final(B, NC, NS):
    NW = NC * NS
    half = B // 2
    per_t = half // (NW // 2)
    G = per_t // 16

    mesh = plsc.VectorSubcoreMesh(core_axis_name="c", subcore_axis_name="s")

    @functools.partial(
        pl.kernel,
        out_type=jax.ShapeDtypeStruct((B,), jnp.float32),
        mesh=mesh,
        scratch_types=(
            pltpu.VMEM((per_t, 16), jnp.float32),
            pltpu.VMEM((per_t * 16,), jnp.float32),
            pltpu.VMEM((per_t,), jnp.float32),
        ),
    )
    def sc_final(fin0, fin1, gmf0, gmf1, out_hbm, finv, gmfv, outv):
        wid = lax.axis_index("s") * NC + lax.axis_index("c")
        lane = jax.lax.iota(jnp.int32, 16)
        perms = [lane ^ k for k in (8, 4, 2, 1)]

        def work(fin_ref, gmf_ref, tid, out_base):
            row0 = tid * per_t
            pltpu.sync_copy(fin_ref.at[pl.ds(row0, per_t)], finv)
            pltpu.sync_copy(gmf_ref.at[pl.ds(row0 * 16, per_t * 16)], gmfv)

            def grp(q, carry):
                acc = jnp.zeros(16, jnp.float32)
                for j in range(16):
                    i = q * 16 + j
                    v = finv[i, :] + gmfv[pl.ds(i * 16, 16)]
                    for p in perms:
                        v = v + jnp.take(v, p)
                    acc = jnp.where(lane == j, v, acc)
                outv[pl.ds(q * 16, 16)] = 1.0 / (1.0 + jnp.exp(-acc))
                return carry

            lax.fori_loop(0, G, grp, 0)
            pltpu.sync_copy(outv, out_hbm.at[pl.ds(out_base + row0, per_t)])

        @pl.when(wid < NW // 2)
        def _():
            work(fin0, gmf0, wid, 0)

        @pl.when(wid >= NW // 2)
        def _():
            work(fin1, gmf1, wid - NW // 2, half)

    return sc_final


def kernel(gmf_user_table, gmf_item_table, mlp_user_table, mlp_item_table,
           W1, b1, W2, b2, W3, b3, Wn, bn, users, items):
    B = users.shape[0]
    D = gmf_user_table.shape[1]
    H1, H2, H3 = W1.shape[1], W2.shape[1], W3.shape[1]

    info = plsc.get_sparse_core_info()
    NC, NS = info.num_cores, info.num_subcores

    users = users.astype(jnp.int32)
    items = items.astype(jnp.int32)
    wn_flat = Wn.reshape(-1)
    wn_m = Wn[D:, :].reshape(1, H3)
    bn16 = jnp.broadcast_to(bn.reshape(1, 1) / H3, (1, H3))

    n_split = 2
    H = B // n_split
    tc = _make_tc_mlp(H, D, H1, H2, H3)

    sc_outs = [_make_sc_gather(H, D, NC, NS, wn_flat.shape[0], i * H)(
                   users, items,
                   gmf_user_table, gmf_item_table,
                   mlp_user_table, mlp_item_table, wn_flat)
               for i in range(n_split)]
    fins = [tc(mu_g, mi_g,
               W1[:D], W1[D:], b1.reshape(1, H1),
               W2, b2.reshape(1, H2),
               W3, b3.reshape(1, H3),
               wn_m, bn16)
            for (mu_g, mi_g, _) in sc_outs]
    fin = _make_sc_final(B, NC, NS)
    return fin(fins[0], fins[1], sc_outs[0][2], sc_outs[1][2])


# speedup vs baseline: 1.0874x; 1.0184x over previous
import functools

import jax
import jax.numpy as jnp
from jax import lax
from jax.experimental import pallas as pl
from jax.experimental.pallas import tpu as pltpu
from jax.experimental.pallas import tpu_sc as plsc


def _make_sc_gather(B, D, NC, NS, WN, off):
    NW = NC * NS
    per_w = B // NW
    CH = 64
    n_ch = per_w // CH
    K8 = D // 16

    mesh = plsc.VectorSubcoreMesh(core_axis_name="c", subcore_axis_name="s")

    def buf_set():
        return (
            pltpu.VMEM((CH, D), jnp.float32),
            pltpu.VMEM((CH, D), jnp.float32),
            pltpu.VMEM((CH, D), jnp.float32),
            pltpu.VMEM((CH, D), jnp.float32),
            pltpu.VMEM((CH, 16), jnp.float32),
            pltpu.SemaphoreType.DMA,
            pltpu.SemaphoreType.DMA,
        )

    @functools.partial(
        pl.kernel,
        out_type=(
            jax.ShapeDtypeStruct((B, D), jnp.float32),
            jax.ShapeDtypeStruct((B, D), jnp.float32),
            jax.ShapeDtypeStruct((B, 16), jnp.float32),
        ),
        mesh=mesh,
        scratch_types=(
            pltpu.VMEM((WN,), jnp.float32),
            pltpu.VMEM((per_w,), jnp.int32),
            pltpu.VMEM((per_w,), jnp.int32),
        ) + buf_set() + buf_set(),
    )
    def sc_gather(users_hbm, items_hbm, gu_t, gi_t, mu_t, mi_t, wn_hbm,
                  mlp_u_out, mlp_i_out, gmfp_out, wnv, uidx_all, iidx_all,
                  *bufs):
        wid = lax.axis_index("s") * NC + lax.axis_index("c")
        pltpu.sync_copy(users_hbm.at[pl.ds(off + wid * per_w, per_w)],
                        uidx_all)
        pltpu.sync_copy(items_hbm.at[pl.ds(off + wid * per_w, per_w)],
                        iidx_all)
        pltpu.sync_copy(wn_hbm, wnv)
        sets = (bufs[:7], bufs[7:])

        def issue(c):
            gu, gi, mu, mi, _, sem_g, _ = sets[c % 2]
            uidx = uidx_all.at[pl.ds(c * CH, CH)]
            iidx = iidx_all.at[pl.ds(c * CH, CH)]
            return (pltpu.async_copy(gu_t.at[uidx], gu, sem_g),
                    pltpu.async_copy(gi_t.at[iidx], gi, sem_g),
                    pltpu.async_copy(mu_t.at[uidx], mu, sem_g),
                    pltpu.async_copy(mi_t.at[iidx], mi, sem_g))

        pending = {0: issue(0)}
        if n_ch > 1:
            pending[1] = issue(1)
        outcps = {}

        for c in range(n_ch):
            gu, gi, mu, mi, gmfp, _, sem_o = sets[c % 2]
            base = wid * per_w + c * CH
            for cp in pending.pop(c):
                cp.wait()
            o1 = pltpu.async_copy(mu, mlp_u_out.at[pl.ds(base, CH)], sem_o)
            o2 = pltpu.async_copy(mi, mlp_i_out.at[pl.ds(base, CH)], sem_o)

            def row_body(i, carry):
                acc = gu[i, pl.ds(0, 16)] * gi[i, pl.ds(0, 16)] * wnv[pl.ds(0, 16)]
                for k in range(1, K8):
                    acc = acc + (gu[i, pl.ds(k * 16, 16)]
                                 * gi[i, pl.ds(k * 16, 16)]
                                 * wnv[pl.ds(k * 16, 16)])
                gmfp[i, :] = acc
                return carry

            lax.fori_loop(0, CH, row_body, 0)
            o3 = pltpu.async_copy(gmfp, gmfp_out.at[pl.ds(base, CH)], sem_o)
            outcps[c] = (o1, o2, o3)

            nxt = c + 2
            if nxt < n_ch:
                for cp in outcps.pop(c):
                    cp.wait()
                pending[nxt] = issue(nxt)

        for c in sorted(outcps):
            for cp in outcps[c]:
                cp.wait()

    return sc_gather


def _tc_mlp_body(mu_ref, mi_ref, gmfp_ref, w1u_ref, w1i_ref, b1_ref,
                 w2_ref, b2_ref, w3_ref, b3_ref, wnm_ref, bn_ref, out_ref):
    h = jnp.dot(mu_ref[...], w1u_ref[...], preferred_element_type=jnp.float32)
    h = h + jnp.dot(mi_ref[...], w1i_ref[...], preferred_element_type=jnp.float32)
    h = jax.nn.relu(h + b1_ref[...])
    h = jax.nn.relu(jnp.dot(h, w2_ref[...], preferred_element_type=jnp.float32)
                    + b2_ref[...])
    h = jax.nn.relu(jnp.dot(h, w3_ref[...], preferred_element_type=jnp.float32)
                    + b3_ref[...])
    logit = jnp.dot(h, wnm_ref[...], preferred_element_type=jnp.float32)
    logit = logit + jnp.sum(gmfp_ref[...], axis=1, keepdims=True) + bn_ref[...]
    out_ref[...] = 1.0 / (1.0 + jnp.exp(-logit))


def _make_tc_mlp(B, D, H1, H2, H3):
    BLK = 2048
    grid = (B // BLK,)
    full = lambda shape: pl.BlockSpec(shape, lambda i: (0, 0))
    return pl.pallas_call(
        _tc_mlp_body,
        grid=grid,
        in_specs=[
            pl.BlockSpec((BLK, D), lambda i: (i, 0)),
            pl.BlockSpec((BLK, D), lambda i: (i, 0)),
            pl.BlockSpec((BLK, 16), lambda i: (i, 0)),
            full((D, H1)),
            full((D, H1)),
            full((1, H1)),
            full((H1, H2)),
            full((1, H2)),
            full((H2, H3)),
            full((1, H3)),
            full((H3, 1)),
            full((1, 1)),
        ],
        out_specs=pl.BlockSpec((BLK, 1), lambda i: (i, 0)),
        out_shape=jax.ShapeDtypeStruct((B, 1), jnp.float32),
    )


def kernel(gmf_user_table, gmf_item_table, mlp_user_table, mlp_item_table,
           W1, b1, W2, b2, W3, b3, Wn, bn, users, items):
    B = users.shape[0]
    D = gmf_user_table.shape[1]
    H1, H2, H3 = W1.shape[1], W2.shape[1], W3.shape[1]

    info = plsc.get_sparse_core_info()
    NC, NS = info.num_cores, info.num_subcores

    users = users.astype(jnp.int32)
    items = items.astype(jnp.int32)
    wn_flat = Wn.reshape(-1)
    wn_m = Wn[D:, :]

    n_split = 2
    H = B // n_split
    tc = _make_tc_mlp(H, D, H1, H2, H3)

    sc_outs = [_make_sc_gather(H, D, NC, NS, wn_flat.shape[0], i * H)(
                   users, items,
                   gmf_user_table, gmf_item_table,
                   mlp_user_table, mlp_item_table, wn_flat)
               for i in range(n_split)]
    preds = [tc(mu_g, mi_g, gmfp,
                W1[:D], W1[D:], b1.reshape(1, H1),
                W2, b2.reshape(1, H2),
                W3, b3.reshape(1, H3),
                wn_m, bn.reshape(1, 1))[:, 0]
             for (mu_g, mi_g, gmfp) in sc_outs]
    return jnp.concatenate(preds, axis=0)


# speedup vs baseline: 1.0944x; 1.0065x over previous
import functools

import jax
import jax.numpy as jnp
from jax import lax
from jax.experimental import pallas as pl
from jax.experimental.pallas import tpu as pltpu
from jax.experimental.pallas import tpu_sc as plsc


def _make_sc_gather(B, D, NC, NS, WN, off):
    NW = NC * NS
    per_w = B // NW
    CH = 64
    n_ch = per_w // CH
    K8 = D // 16

    mesh = plsc.VectorSubcoreMesh(core_axis_name="c", subcore_axis_name="s")

    def buf_set():
        return (
            pltpu.VMEM((CH, D), jnp.float32),
            pltpu.VMEM((CH, D), jnp.float32),
            pltpu.VMEM((CH, D), jnp.float32),
            pltpu.VMEM((CH, D), jnp.float32),
            pltpu.VMEM((CH, 16), jnp.float32),
            pltpu.SemaphoreType.DMA,
            pltpu.SemaphoreType.DMA,
        )

    @functools.partial(
        pl.kernel,
        out_type=(
            jax.ShapeDtypeStruct((B, D), jnp.float32),
            jax.ShapeDtypeStruct((B, D), jnp.float32),
            jax.ShapeDtypeStruct((B, 16), jnp.float32),
        ),
        mesh=mesh,
        scratch_types=(
            pltpu.VMEM((WN,), jnp.float32),
            pltpu.VMEM((per_w,), jnp.int32),
            pltpu.VMEM((per_w,), jnp.int32),
        ) + buf_set() + buf_set(),
    )
    def sc_gather(users_hbm, items_hbm, gu_t, gi_t, mu_t, mi_t, wn_hbm,
                  mlp_u_out, mlp_i_out, gmfp_out, wnv, uidx_all, iidx_all,
                  *bufs):
        wid = lax.axis_index("s") * NC + lax.axis_index("c")
        pltpu.sync_copy(users_hbm.at[pl.ds(off + wid * per_w, per_w)],
                        uidx_all)
        pltpu.sync_copy(items_hbm.at[pl.ds(off + wid * per_w, per_w)],
                        iidx_all)
        pltpu.sync_copy(wn_hbm, wnv)
        sets = (bufs[:7], bufs[7:])

        def issue(c):
            gu, gi, mu, mi, _, sem_g, _ = sets[c % 2]
            uidx = uidx_all.at[pl.ds(c * CH, CH)]
            iidx = iidx_all.at[pl.ds(c * CH, CH)]
            return (pltpu.async_copy(gu_t.at[uidx], gu, sem_g),
                    pltpu.async_copy(gi_t.at[iidx], gi, sem_g),
                    pltpu.async_copy(mu_t.at[uidx], mu, sem_g),
                    pltpu.async_copy(mi_t.at[iidx], mi, sem_g))

        pending = {0: issue(0)}
        if n_ch > 1:
            pending[1] = issue(1)
        outcps = {}

        for c in range(n_ch):
            gu, gi, mu, mi, gmfp, _, sem_o = sets[c % 2]
            base = wid * per_w + c * CH
            for cp in pending.pop(c):
                cp.wait()
            o1 = pltpu.async_copy(mu, mlp_u_out.at[pl.ds(base, CH)], sem_o)
            o2 = pltpu.async_copy(mi, mlp_i_out.at[pl.ds(base, CH)], sem_o)

            def row_body(i, carry):
                acc = gu[i, pl.ds(0, 16)] * gi[i, pl.ds(0, 16)] * wnv[pl.ds(0, 16)]
                for k in range(1, K8):
                    acc = acc + (gu[i, pl.ds(k * 16, 16)]
                                 * gi[i, pl.ds(k * 16, 16)]
                                 * wnv[pl.ds(k * 16, 16)])
                gmfp[i, :] = acc
                return carry

            lax.fori_loop(0, CH, row_body, 0)
            o3 = pltpu.async_copy(gmfp, gmfp_out.at[pl.ds(base, CH)], sem_o)
            outcps[c] = (o1, o2, o3)

            nxt = c + 2
            if nxt < n_ch:
                for cp in outcps.pop(c):
                    cp.wait()
                pending[nxt] = issue(nxt)

        for c in sorted(outcps):
            for cp in outcps[c]:
                cp.wait()

    return sc_gather


def _tc_mlp_body(mu_ref, mi_ref, gmfp_ref, w1u_ref, w1i_ref, b1_ref,
                 w2_ref, b2_ref, w3_ref, b3_ref, wnm_ref, bn_ref, out_ref):
    h = jnp.dot(mu_ref[...], w1u_ref[...], preferred_element_type=jnp.float32)
    h = h + jnp.dot(mi_ref[...], w1i_ref[...], preferred_element_type=jnp.float32)
    h = jax.nn.relu(h + b1_ref[...])
    h = jax.nn.relu(jnp.dot(h, w2_ref[...], preferred_element_type=jnp.float32)
                    + b2_ref[...])
    h = jax.nn.relu(jnp.dot(h, w3_ref[...], preferred_element_type=jnp.float32)
                    + b3_ref[...])
    logit = jnp.dot(h, wnm_ref[...], preferred_element_type=jnp.float32)
    logit = logit + jnp.sum(gmfp_ref[...], axis=1, keepdims=True) + bn_ref[...]
    out_ref[...] = 1.0 / (1.0 + jnp.exp(-logit))


def _make_tc_mlp(B, D, H1, H2, H3):
    BLK = 2048
    grid = (B // BLK,)
    assert B % BLK == 0
    full = lambda shape: pl.BlockSpec(shape, lambda i: (0, 0))
    return pl.pallas_call(
        _tc_mlp_body,
        grid=grid,
        in_specs=[
            pl.BlockSpec((BLK, D), lambda i: (i, 0)),
            pl.BlockSpec((BLK, D), lambda i: (i, 0)),
            pl.BlockSpec((BLK, 16), lambda i: (i, 0)),
            full((D, H1)),
            full((D, H1)),
            full((1, H1)),
            full((H1, H2)),
            full((1, H2)),
            full((H2, H3)),
            full((1, H3)),
            full((H3, 1)),
            full((1, 1)),
        ],
        out_specs=pl.BlockSpec((BLK, 1), lambda i: (i, 0)),
        out_shape=jax.ShapeDtypeStruct((B, 1), jnp.float32),
    )


def kernel(gmf_user_table, gmf_item_table, mlp_user_table, mlp_item_table,
           W1, b1, W2, b2, W3, b3, Wn, bn, users, items):
    B = users.shape[0]
    D = gmf_user_table.shape[1]
    H1, H2, H3 = W1.shape[1], W2.shape[1], W3.shape[1]

    info = plsc.get_sparse_core_info()
    NC, NS = info.num_cores, info.num_subcores

    users = users.astype(jnp.int32)
    items = items.astype(jnp.int32)
    wn_flat = Wn.reshape(-1)
    wn_m = Wn[D:, :]

    splits = [(0, B * 5 // 8), (B * 5 // 8, B * 3 // 8)]
    sc_outs = [_make_sc_gather(n, D, NC, NS, wn_flat.shape[0], off)(
                   users, items,
                   gmf_user_table, gmf_item_table,
                   mlp_user_table, mlp_item_table, wn_flat)
               for off, n in splits]
    preds = [_make_tc_mlp(n, D, H1, H2, H3)(
                 mu_g, mi_g, gmfp,
                 W1[:D], W1[D:], b1.reshape(1, H1),
                 W2, b2.reshape(1, H2),
                 W3, b3.reshape(1, H3),
                 wn_m, bn.reshape(1, 1))[:, 0]
             for (_, n), (mu_g, mi_g, gmfp) in zip(splits, sc_outs)]
    return jnp.concatenate(preds, axis=0)


# speedup vs baseline: 1.1205x; 1.0239x over previous
import functools

import jax
import jax.numpy as jnp
from jax import lax
from jax.experimental import pallas as pl
from jax.experimental.pallas import tpu as pltpu
from jax.experimental.pallas import tpu_sc as plsc


def _make_sc_gather(B, D, NC, NS, WN, off):
    NW = NC * NS
    per_w = B // NW
    CH = 64
    n_ch = per_w // CH
    K8 = D // 16

    mesh = plsc.VectorSubcoreMesh(core_axis_name="c", subcore_axis_name="s")

    def buf_set():
        return (
            pltpu.VMEM((CH, D), jnp.float32),
            pltpu.VMEM((CH, D), jnp.float32),
            pltpu.VMEM((CH, D), jnp.float32),
            pltpu.VMEM((CH, D), jnp.float32),
            pltpu.VMEM((CH, 16), jnp.float32),
            pltpu.SemaphoreType.DMA,
            pltpu.SemaphoreType.DMA,
        )

    @functools.partial(
        pl.kernel,
        out_type=(
            jax.ShapeDtypeStruct((B, D), jnp.float32),
            jax.ShapeDtypeStruct((B, D), jnp.float32),
            jax.ShapeDtypeStruct((B, 16), jnp.float32),
        ),
        mesh=mesh,
        scratch_types=(
            pltpu.VMEM((WN,), jnp.float32),
            pltpu.VMEM((per_w,), jnp.int32),
            pltpu.VMEM((per_w,), jnp.int32),
            pltpu.SemaphoreType.DMA,
        ) + buf_set() + buf_set(),
    )
    def sc_gather(users_hbm, items_hbm, gu_t, gi_t, mu_t, mi_t, wn_hbm,
                  mlp_u_out, mlp_i_out, gmfp_out, wnv, uidx_all, iidx_all,
                  sem_i, *bufs):
        wid = lax.axis_index("s") * NC + lax.axis_index("c")
        cp_u = pltpu.async_copy(users_hbm.at[pl.ds(off + wid * per_w, per_w)],
                                uidx_all, sem_i)
        cp_i = pltpu.async_copy(items_hbm.at[pl.ds(off + wid * per_w, per_w)],
                                iidx_all, sem_i)
        cp_w = pltpu.async_copy(wn_hbm, wnv, sem_i)
        cp_u.wait()
        cp_i.wait()
        sets = (bufs[:7], bufs[7:])

        def issue(c):
            gu, gi, mu, mi, _, sem_g, _ = sets[c % 2]
            uidx = uidx_all.at[pl.ds(c * CH, CH)]
            iidx = iidx_all.at[pl.ds(c * CH, CH)]
            return (pltpu.async_copy(gu_t.at[uidx], gu, sem_g),
                    pltpu.async_copy(gi_t.at[iidx], gi, sem_g),
                    pltpu.async_copy(mu_t.at[uidx], mu, sem_g),
                    pltpu.async_copy(mi_t.at[iidx], mi, sem_g))

        pending = {0: issue(0)}
        if n_ch > 1:
            pending[1] = issue(1)
        cp_w.wait()
        outcps = {}

        for c in range(n_ch):
            gu, gi, mu, mi, gmfp, _, sem_o = sets[c % 2]
            base = wid * per_w + c * CH
            for cp in pending.pop(c):
                cp.wait()
            o1 = pltpu.async_copy(mu, mlp_u_out.at[pl.ds(base, CH)], sem_o)
            o2 = pltpu.async_copy(mi, mlp_i_out.at[pl.ds(base, CH)], sem_o)

            def row_body(i, carry):
                acc = gu[i, pl.ds(0, 16)] * gi[i, pl.ds(0, 16)] * wnv[pl.ds(0, 16)]
                for k in range(1, K8):
                    acc = acc + (gu[i, pl.ds(k * 16, 16)]
                                 * gi[i, pl.ds(k * 16, 16)]
                                 * wnv[pl.ds(k * 16, 16)])
                gmfp[i, :] = acc
                return carry

            lax.fori_loop(0, CH, row_body, 0)
            o3 = pltpu.async_copy(gmfp, gmfp_out.at[pl.ds(base, CH)], sem_o)
            outcps[c] = (o1, o2, o3)

            nxt = c + 2
            if nxt < n_ch:
                for cp in outcps.pop(c):
                    cp.wait()
                pending[nxt] = issue(nxt)

        for c in sorted(outcps):
            for cp in outcps[c]:
                cp.wait()

    return sc_gather


def _tc_mlp_body(mu_ref, mi_ref, gmfp_ref, w1u_ref, w1i_ref, b1_ref,
                 w2_ref, b2_ref, w3_ref, b3_ref, wnm_ref, bn_ref, out_ref):
    h = jnp.dot(mu_ref[...], w1u_ref[...], preferred_element_type=jnp.float32)
    h = h + jnp.dot(mi_ref[...], w1i_ref[...], preferred_element_type=jnp.float32)
    h = jax.nn.relu(h + b1_ref[...])
    h = jax.nn.relu(jnp.dot(h, w2_ref[...], preferred_element_type=jnp.float32)
                    + b2_ref[...])
    h = jax.nn.relu(jnp.dot(h, w3_ref[...], preferred_element_type=jnp.float32)
                    + b3_ref[...])
    logit = jnp.dot(h, wnm_ref[...], preferred_element_type=jnp.float32)
    logit = logit + jnp.sum(gmfp_ref[...], axis=1, keepdims=True) + bn_ref[...]
    out_ref[...] = 1.0 / (1.0 + jnp.exp(-logit))


def _make_tc_mlp(B, D, H1, H2, H3):
    BLK = 2048
    grid = (B // BLK,)
    assert B % BLK == 0
    full = lambda shape: pl.BlockSpec(shape, lambda i: (0, 0))
    return pl.pallas_call(
        _tc_mlp_body,
        grid=grid,
        in_specs=[
            pl.BlockSpec((BLK, D), lambda i: (i, 0)),
            pl.BlockSpec((BLK, D), lambda i: (i, 0)),
            pl.BlockSpec((BLK, 16), lambda i: (i, 0)),
            full((D, H1)),
            full((D, H1)),
            full((1, H1)),
            full((H1, H2)),
            full((1, H2)),
            full((H2, H3)),
            full((1, H3)),
            full((H3, 1)),
            full((1, 1)),
        ],
        out_specs=pl.BlockSpec((BLK, 1), lambda i: (i, 0)),
        out_shape=jax.ShapeDtypeStruct((B, 1), jnp.float32),
    )


def kernel(gmf_user_table, gmf_item_table, mlp_user_table, mlp_item_table,
           W1, b1, W2, b2, W3, b3, Wn, bn, users, items):
    B = users.shape[0]
    D = gmf_user_table.shape[1]
    H1, H2, H3 = W1.shape[1], W2.shape[1], W3.shape[1]

    info = plsc.get_sparse_core_info()
    NC, NS = info.num_cores, info.num_subcores

    users = users.astype(jnp.int32)
    items = items.astype(jnp.int32)
    wn_flat = Wn.reshape(-1)
    wn_m = Wn[D:, :]

    splits = [(0, B * 5 // 8), (B * 5 // 8, B * 3 // 8)]
    sc_outs = [_make_sc_gather(n, D, NC, NS, wn_flat.shape[0], off)(
                   users, items,
                   gmf_user_table, gmf_item_table,
                   mlp_user_table, mlp_item_table, wn_flat)
               for off, n in splits]
    preds = [_make_tc_mlp(n, D, H1, H2, H3)(
                 mu_g, mi_g, gmfp,
                 W1[:D], W1[D:], b1.reshape(1, H1),
                 W2, b2.reshape(1, H2),
                 W3, b3.reshape(1, H3),
                 wn_m, bn.reshape(1, 1))[:, 0]
             for (_, n), (mu_g, mi_g, gmfp) in zip(splits, sc_outs)]
    return jnp.concatenate(preds, axis=0)


# speedup vs baseline: 1.1215x; 1.0008x over previous
import functools

import jax
import jax.numpy as jnp
from jax import lax
from jax.experimental import pallas as pl
from jax.experimental.pallas import tpu as pltpu
from jax.experimental.pallas import tpu_sc as plsc


def _make_sc_gather(B, D, NC, NS, WN, off):
    NW = NC * NS
    per_w = B // NW
    CH = 64
    n_ch = per_w // CH
    K8 = D // 16

    mesh = plsc.VectorSubcoreMesh(core_axis_name="c", subcore_axis_name="s")

    def buf_set():
        return (
            pltpu.VMEM((CH, D), jnp.float32),
            pltpu.VMEM((CH, D), jnp.float32),
            pltpu.VMEM((CH, D), jnp.float32),
            pltpu.VMEM((CH, D), jnp.float32),
            pltpu.VMEM((CH, 16), jnp.float32),
            pltpu.SemaphoreType.DMA,
            pltpu.SemaphoreType.DMA,
        )

    @functools.partial(
        pl.kernel,
        out_type=(
            jax.ShapeDtypeStruct((B, D), jnp.float32),
            jax.ShapeDtypeStruct((B, D), jnp.float32),
            jax.ShapeDtypeStruct((B, 16), jnp.float32),
        ),
        mesh=mesh,
        scratch_types=(
            pltpu.VMEM((WN,), jnp.float32),
            pltpu.VMEM((per_w,), jnp.int32),
            pltpu.VMEM((per_w,), jnp.int32),
            pltpu.SemaphoreType.DMA,
        ) + buf_set() + buf_set() + buf_set(),
    )
    def sc_gather(users_hbm, items_hbm, gu_t, gi_t, mu_t, mi_t, wn_hbm,
                  mlp_u_out, mlp_i_out, gmfp_out, wnv, uidx_all, iidx_all,
                  sem_i, *bufs):
        wid = lax.axis_index("s") * NC + lax.axis_index("c")
        cp_u = pltpu.async_copy(users_hbm.at[pl.ds(off + wid * per_w, per_w)],
                                uidx_all, sem_i)
        cp_i = pltpu.async_copy(items_hbm.at[pl.ds(off + wid * per_w, per_w)],
                                iidx_all, sem_i)
        cp_w = pltpu.async_copy(wn_hbm, wnv, sem_i)
        cp_u.wait()
        cp_i.wait()
        sets = (bufs[:7], bufs[7:14], bufs[14:])

        def issue(c):
            gu, gi, mu, mi, _, sem_g, _ = sets[c % 3]
            uidx = uidx_all.at[pl.ds(c * CH, CH)]
            iidx = iidx_all.at[pl.ds(c * CH, CH)]
            return (pltpu.async_copy(gu_t.at[uidx], gu, sem_g),
                    pltpu.async_copy(gi_t.at[iidx], gi, sem_g),
                    pltpu.async_copy(mu_t.at[uidx], mu, sem_g),
                    pltpu.async_copy(mi_t.at[iidx], mi, sem_g))

        pending = {c: issue(c) for c in range(min(3, n_ch))}
        cp_w.wait()
        outcps = {}

        for c in range(n_ch):
            gu, gi, mu, mi, gmfp, _, sem_o = sets[c % 3]
            base = wid * per_w + c * CH
            for cp in pending.pop(c):
                cp.wait()
            o1 = pltpu.async_copy(mu, mlp_u_out.at[pl.ds(base, CH)], sem_o)
            o2 = pltpu.async_copy(mi, mlp_i_out.at[pl.ds(base, CH)], sem_o)

            def row_body(i, carry):
                acc = gu[i, pl.ds(0, 16)] * gi[i, pl.ds(0, 16)] * wnv[pl.ds(0, 16)]
                for k in range(1, K8):
                    acc = acc + (gu[i, pl.ds(k * 16, 16)]
                                 * gi[i, pl.ds(k * 16, 16)]
                                 * wnv[pl.ds(k * 16, 16)])
                gmfp[i, :] = acc
                return carry

            lax.fori_loop(0, CH, row_body, 0)
            o3 = pltpu.async_copy(gmfp, gmfp_out.at[pl.ds(base, CH)], sem_o)
            outcps[c] = (o1, o2, o3)

            nxt = c + 3
            if nxt < n_ch:
                for cp in outcps.pop(c):
                    cp.wait()
                pending[nxt] = issue(nxt)

        for c in sorted(outcps):
            for cp in outcps[c]:
                cp.wait()

    return sc_gather


def _tc_mlp_body(mu_ref, mi_ref, gmfp_ref, w1u_ref, w1i_ref, b1_ref,
                 w2_ref, b2_ref, w3_ref, b3_ref, wnm_ref, bn_ref, out_ref):
    h = jnp.dot(mu_ref[...], w1u_ref[...], preferred_element_type=jnp.float32)
    h = h + jnp.dot(mi_ref[...], w1i_ref[...], preferred_element_type=jnp.float32)
    h = jax.nn.relu(h + b1_ref[...])
    h = jax.nn.relu(jnp.dot(h, w2_ref[...], preferred_element_type=jnp.float32)
                    + b2_ref[...])
    h = jax.nn.relu(jnp.dot(h, w3_ref[...], preferred_element_type=jnp.float32)
                    + b3_ref[...])
    logit = jnp.dot(h, wnm_ref[...], preferred_element_type=jnp.float32)
    logit = logit + jnp.sum(gmfp_ref[...], axis=1, keepdims=True) + bn_ref[...]
    out_ref[...] = 1.0 / (1.0 + jnp.exp(-logit))


def _make_tc_mlp(B, D, H1, H2, H3):
    BLK = 2048
    grid = (B // BLK,)
    assert B % BLK == 0
    full = lambda shape: pl.BlockSpec(shape, lambda i: (0, 0))
    return pl.pallas_call(
        _tc_mlp_body,
        grid=grid,
        in_specs=[
            pl.BlockSpec((BLK, D), lambda i: (i, 0)),
            pl.BlockSpec((BLK, D), lambda i: (i, 0)),
            pl.BlockSpec((BLK, 16), lambda i: (i, 0)),
            full((D, H1)),
            full((D, H1)),
            full((1, H1)),
            full((H1, H2)),
            full((1, H2)),
            full((H2, H3)),
            full((1, H3)),
            full((H3, 1)),
            full((1, 1)),
        ],
        out_specs=pl.BlockSpec((BLK, 1), lambda i: (i, 0)),
        out_shape=jax.ShapeDtypeStruct((B, 1), jnp.float32),
    )


def kernel(gmf_user_table, gmf_item_table, mlp_user_table, mlp_item_table,
           W1, b1, W2, b2, W3, b3, Wn, bn, users, items):
    B = users.shape[0]
    D = gmf_user_table.shape[1]
    H1, H2, H3 = W1.shape[1], W2.shape[1], W3.shape[1]

    info = plsc.get_sparse_core_info()
    NC, NS = info.num_cores, info.num_subcores

    users = users.astype(jnp.int32)
    items = items.astype(jnp.int32)
    wn_flat = Wn.reshape(-1)
    wn_m = Wn[D:, :]

    splits = [(0, B * 5 // 8), (B * 5 // 8, B * 3 // 8)]
    sc_outs = [_make_sc_gather(n, D, NC, NS, wn_flat.shape[0], off)(
                   users, items,
                   gmf_user_table, gmf_item_table,
                   mlp_user_table, mlp_item_table, wn_flat)
               for off, n in splits]
    preds = [_make_tc_mlp(n, D, H1, H2, H3)(
                 mu_g, mi_g, gmfp,
                 W1[:D], W1[D:], b1.reshape(1, H1),
                 W2, b2.reshape(1, H2),
                 W3, b3.reshape(1, H3),
                 wn_m, bn.reshape(1, 1))[:, 0]
             for (_, n), (mu_g, mi_g, gmfp) in zip(splits, sc_outs)]
    return jnp.concatenate(preds, axis=0)
